# baseline (speedup 1.0000x reference)
import jax
import jax.numpy as jnp
import numpy as np
from jax.experimental import pallas as pl
from jax.experimental.pallas import tpu as pltpu

B, D, C, P = 2048, 128, 200, 200
N = B + P
TILE = 768
NP_ = 2304
NT = NP_ // TILE
KA = 136
MARGIN = 1.0
N_PAIRS = float(N * (N - 1) // 2)

_PAIRS = np.array([(i, j) for i in range(NT) for j in range(i, NT)],
                  dtype=np.int32).T
NUM_TILES = _PAIRS.shape[1]

_PAD_VALS = (10.0 * (np.arange(NP_ - N, dtype=np.float32) + 1.0))
_PAD_EMB = np.broadcast_to(_PAD_VALS[:, None], (NP_ - N, D)).copy()
_PAD_LAB = (-(np.arange(NP_ - N, dtype=np.int32) + 1))


def _argmax_body(lab_ref, out_ref):
    v = lab_ref[...]
    m = jnp.max(v, axis=1, keepdims=True)
    iota = jax.lax.broadcasted_iota(jnp.int32, v.shape, 1)
    idx = jnp.min(jnp.where(v == m, iota, C), axis=1, keepdims=True)
    out_ref[...] = idx


def _prep_body(x_ref, xt_ref, aug_ref, augt_ref):
    x = x_ref[...]
    xt = xt_ref[...]
    sq_c = jnp.sum(x * x, axis=1, keepdims=True)
    sq_r = jnp.sum(xt * xt, axis=0, keepdims=True)
    ones_c = jnp.ones((TILE, 1), jnp.float32)
    aug_ref[...] = jnp.concatenate(
        [x, ones_c, sq_c, jnp.zeros((TILE, KA - D - 2), jnp.float32)], axis=1)
    augt_ref[...] = jnp.concatenate(
        [-2.0 * xt, sq_r, jnp.ones((1, TILE), jnp.float32),
         jnp.zeros((KA - D - 2, TILE), jnp.float32)], axis=0)


def _loss_body(tiles_ref, ai_ref, bjt_ref, li_ref, lj_ref, out_ref):
    t = pl.program_id(0)

    @pl.when(t == 0)
    def _init():
        out_ref[0, 0] = 0.0

    d2raw = jax.lax.dot_general(ai_ref[...], bjt_ref[...],
                                (((1,), (0,)), ((), ())),
                                preferred_element_type=jnp.float32)
    d2 = jnp.maximum(d2raw, 0.0)
    r = MARGIN - jnp.sqrt(jnp.minimum(d2, MARGIN * MARGIN))
    same = li_ref[...] == lj_ref[...]
    base = jnp.where(same, d2, r * r)
    s = jnp.sum(base)

    diag = tiles_ref[0, t] == tiles_ref[1, t]
    scale = jnp.where(diag, 0.5, 1.0)
    out_ref[0, 0] += s * scale

    @pl.when(t == NUM_TILES - 1)
    def _finish():
        out_ref[0, 0] = out_ref[0, 0] * (1.0 / N_PAIRS)


def kernel(embeddings, labels, prototypes, proto_keys):
    labels_p = jnp.pad(labels, ((0, 0), (0, 256 - C)),
                       constant_values=-np.inf)
    emb_p = jnp.concatenate(
        [embeddings, prototypes, jnp.asarray(_PAD_EMB)], axis=0)
    emb_t = emb_p.T

    lab_col = pl.pallas_call(
        _argmax_body,
        grid=(B // 256,),
        in_specs=[pl.BlockSpec((256, 256), lambda i: (i, 0))],
        out_specs=pl.BlockSpec((256, 1), lambda i: (i, 0)),
        out_shape=jax.ShapeDtypeStruct((B, 1), jnp.int32),
    )(labels_p)

    lab_all = jnp.concatenate(
        [lab_col[:, 0], proto_keys.astype(jnp.int32), jnp.asarray(_PAD_LAB)])
    lab_c = lab_all[:, None]
    lab_r = lab_all[None, :]

    aug, augt = pl.pallas_call(
        _prep_body,
        grid=(NT,),
        in_specs=[
            pl.BlockSpec((TILE, D), lambda i: (i, 0)),
            pl.BlockSpec((D, TILE), lambda i: (0, i)),
        ],
        out_specs=[
            pl.BlockSpec((TILE, KA), lambda i: (i, 0)),
            pl.BlockSpec((KA, TILE), lambda i: (0, i)),
        ],
        out_shape=[
            jax.ShapeDtypeStruct((NP_, KA), jnp.float32),
            jax.ShapeDtypeStruct((KA, NP_), jnp.float32),
        ],
    )(emb_p, emb_t)

    tiles = jnp.asarray(_PAIRS)

    out = pl.pallas_call(
        _loss_body,
        grid_spec=pltpu.PrefetchScalarGridSpec(
            num_scalar_prefetch=1,
            grid=(NUM_TILES,),
            in_specs=[
                pl.BlockSpec((TILE, KA), lambda t, tiles: (tiles[0, t], 0)),
                pl.BlockSpec((KA, TILE), lambda t, tiles: (0, tiles[1, t])),
                pl.BlockSpec((TILE, 1), lambda t, tiles: (tiles[0, t], 0)),
                pl.BlockSpec((1, TILE), lambda t, tiles: (0, tiles[1, t])),
            ],
            out_specs=pl.BlockSpec(memory_space=pltpu.SMEM),
        ),
        out_shape=jax.ShapeDtypeStruct((1, 1), jnp.float32),
    )(tiles, aug, augt, lab_c, lab_r)
    return out[0, 0]


# speedup vs baseline: 1.1604x; 1.1604x over previous
import jax
import jax.numpy as jnp
import numpy as np
from jax.experimental import pallas as pl
from jax.experimental.pallas import tpu as pltpu

B, D, C, P = 2048, 128, 200, 200
N = B + P
TILE = 768
NP_ = 2304
NT = NP_ // TILE
MARGIN = 1.0
N_PAIRS = float(N * (N - 1) // 2)

_PAIRS = np.array([(i, j) for i in range(NT) for j in range(i, NT)],
                  dtype=np.int32).T
NUM_TILES = _PAIRS.shape[1]

_PAD_VALS = (2.0 * (np.arange(NP_ - N, dtype=np.float32) + 1.0))
_PAD_EMB = np.broadcast_to(_PAD_VALS[:, None], (NP_ - N, D)).copy()
_PAD_LAB = (-(np.arange(NP_ - N, dtype=np.int32) + 1))


def _argmax_body(lab_ref, out_ref):
    v = lab_ref[...]
    m = jnp.max(v, axis=1, keepdims=True)
    iota = jax.lax.broadcasted_iota(jnp.int32, v.shape, 1)
    idx = jnp.min(jnp.where(v == m, iota, C), axis=1, keepdims=True)
    out_ref[...] = idx


def _loss_body(tiles_ref, xi_ref, xjt_ref, li_ref, lj_ref, out_ref):
    t = pl.program_id(0)

    @pl.when(t == 0)
    def _init():
        out_ref[0, 0] = 0.0

    xi = xi_ref[...]
    xjt = xjt_ref[...]
    xjt2 = xjt + xjt
    dot2 = jax.lax.dot_general(xi, xjt2, (((1,), (0,)), ((), ())),
                               preferred_element_type=jnp.float32)
    sqi = jnp.sum(xi * xi, axis=1, keepdims=True)
    sqj = jnp.sum(xjt * xjt, axis=0, keepdims=True)
    d2 = jnp.maximum((sqi + sqj) - dot2, 0.0)
    r = MARGIN - jnp.sqrt(jnp.minimum(d2, MARGIN * MARGIN))
    same = li_ref[...] == lj_ref[...]
    base = jnp.where(same, d2, r * r)
    s = jnp.sum(base)

    diag = tiles_ref[0, t] == tiles_ref[1, t]
    scale = jnp.where(diag, 0.5, 1.0)
    out_ref[0, 0] += s * scale

    @pl.when(t == NUM_TILES - 1)
    def _finish():
        out_ref[0, 0] = out_ref[0, 0] * (1.0 / N_PAIRS)


def kernel(embeddings, labels, prototypes, proto_keys):
    labels_p = jnp.pad(labels, ((0, 0), (0, 256 - C)),
                       constant_values=-np.inf)
    emb_p = jnp.concatenate(
        [embeddings, prototypes, jnp.asarray(_PAD_EMB)], axis=0)
    emb_t = emb_p.T

    lab_col = pl.pallas_call(
        _argmax_body,
        grid=(B // 256,),
        in_specs=[pl.BlockSpec((256, 256), lambda i: (i, 0))],
        out_specs=pl.BlockSpec((256, 1), lambda i: (i, 0)),
        out_shape=jax.ShapeDtypeStruct((B, 1), jnp.int32),
    )(labels_p)

    lab_all = jnp.concatenate(
        [lab_col[:, 0], proto_keys.astype(jnp.int32), jnp.asarray(_PAD_LAB)])
    lab_c = lab_all[:, None]
    lab_r = lab_all[None, :]

    tiles = jnp.asarray(_PAIRS)

    out = pl.pallas_call(
        _loss_body,
        grid_spec=pltpu.PrefetchScalarGridSpec(
            num_scalar_prefetch=1,
            grid=(NUM_TILES,),
            in_specs=[
                pl.BlockSpec((TILE, D), lambda t, tiles: (tiles[0, t], 0)),
                pl.BlockSpec((D, TILE), lambda t, tiles: (0, tiles[1, t])),
                pl.BlockSpec((TILE, 1), lambda t, tiles: (tiles[0, t], 0)),
                pl.BlockSpec((1, TILE), lambda t, tiles: (0, tiles[1, t])),
            ],
            out_specs=pl.BlockSpec(memory_space=pltpu.SMEM),
        ),
        out_shape=jax.ShapeDtypeStruct((1, 1), jnp.float32),
    )(tiles, emb_p, emb_t, lab_c, lab_r)
    return out[0, 0]


# speedup vs baseline: 1.7477x; 1.5060x over previous
import jax
import jax.numpy as jnp
import numpy as np
from jax.experimental import pallas as pl
from jax.experimental.pallas import tpu as pltpu

B, D, C, P = 2048, 128, 200, 200
N = B + P
TILE = 768
NP_ = 2304
NT = NP_ // TILE
NPAD = NP_ - N
MARGIN = 1.0
N_PAIRS = float(N * (N - 1) // 2)

_PAIRS = np.array([(i, j) for i in range(NT) for j in range(i, NT)],
                  dtype=np.int32).T
NUM_TILES = _PAIRS.shape[1]


def _body(tiles_ref, emb_ref, lab_ref, proto_ref, pk_ref,
          out_ref, xall, sq_all, lab_c):
    t = pl.program_id(0)

    @pl.when(t == 0)
    def _prep():
        out_ref[0, 0] = 0.0
        xall[0:B, :] = emb_ref[...]
        xall[B:N, :] = proto_ref[...]
        padv = 2.0 * (jax.lax.broadcasted_iota(jnp.int32, (NPAD, D), 0)
                      .astype(jnp.float32) + 1.0)
        xall[N:NP_, :] = padv
        x = xall[...]
        sq_all[...] = jnp.sum(x * x, axis=1, keepdims=True)
        v = lab_ref[...]
        m = jnp.max(v, axis=1, keepdims=True)
        iota = jax.lax.broadcasted_iota(jnp.int32, v.shape, 1)
        lab_c[0:B, :] = jnp.min(jnp.where(v == m, iota, C), axis=1,
                                keepdims=True)
        lab_c[B:N, :] = pk_ref[...]
        lab_c[N:NP_, :] = -(jax.lax.broadcasted_iota(jnp.int32, (NPAD, 1), 0)
                            + 1)

    bi = tiles_ref[0, t]
    bj = tiles_ref[1, t]
    ri = pl.ds(bi * TILE, TILE)
    rj = pl.ds(bj * TILE, TILE)

    xi = xall[ri, :]
    xj = xall[rj, :]
    dot2 = jax.lax.dot_general(xi, xj + xj, (((1,), (1,)), ((), ())),
                               preferred_element_type=jnp.float32)
    sqi = sq_all[ri, :]
    sqj = jnp.transpose(sq_all[rj, :])
    d2 = jnp.maximum((sqi + sqj) - dot2, 0.0)
    r = MARGIN - jnp.sqrt(jnp.minimum(d2, MARGIN * MARGIN))
    same = lab_c[ri, :] == jnp.transpose(lab_c[rj, :])
    base = jnp.where(same, d2, r * r)
    s = jnp.sum(base)

    scale = jnp.where(bi == bj, 0.5, 1.0)
    out_ref[0, 0] += s * scale

    @pl.when(t == NUM_TILES - 1)
    def _finish():
        out_ref[0, 0] = out_ref[0, 0] * (1.0 / N_PAIRS)


def kernel(embeddings, labels, prototypes, proto_keys):
    tiles = jnp.asarray(_PAIRS)
    pk2d = proto_keys.astype(jnp.int32)[:, None]

    out = pl.pallas_call(
        _body,
        grid_spec=pltpu.PrefetchScalarGridSpec(
            num_scalar_prefetch=1,
            grid=(NUM_TILES,),
            in_specs=[
                pl.BlockSpec((B, D), lambda t, tiles: (0, 0)),
                pl.BlockSpec((B, C), lambda t, tiles: (0, 0)),
                pl.BlockSpec((P, D), lambda t, tiles: (0, 0)),
                pl.BlockSpec((P, 1), lambda t, tiles: (0, 0)),
            ],
            out_specs=pl.BlockSpec(memory_space=pltpu.SMEM),
            scratch_shapes=[
                pltpu.VMEM((NP_, D), jnp.float32),
                pltpu.VMEM((NP_, 1), jnp.float32),
                pltpu.VMEM((NP_, 1), jnp.int32),
            ],
        ),
        out_shape=jax.ShapeDtypeStruct((1, 1), jnp.float32),
    )(tiles, embeddings, labels, prototypes, pk2d)
    return out[0, 0]


# speedup vs baseline: 1.9255x; 1.1018x over previous
import jax
import jax.numpy as jnp
import numpy as np
from jax.experimental import pallas as pl
from jax.experimental.pallas import tpu as pltpu

B, D, C, P = 2048, 128, 200, 200
N = B + P
TILE = 768
NP_ = 2304
NT = NP_ // TILE
NPAD = NP_ - N
MARGIN = 1.0
N_PAIRS = float(N * (N - 1) // 2)

_PAIRS = np.array([(i, j) for i in range(NT) for j in range(i, NT)],
                  dtype=np.int32).T
NUM_TILES = _PAIRS.shape[1]


def _body(tiles_ref, emb_ref, lab_ref, proto_ref, pk_ref,
          out_ref, xall, x2all, sq_c, sq_r, lab_c, lab_r):
    t = pl.program_id(0)

    @pl.when(t == 0)
    def _prep():
        out_ref[0, 0] = 0.0
        xall[0:B, :] = emb_ref[...]
        xall[B:N, :] = proto_ref[...]
        padv = 2.0 * (jax.lax.broadcasted_iota(jnp.int32, (NPAD, D), 0)
                      .astype(jnp.float32) + 1.0)
        xall[N:NP_, :] = padv
        x = xall[...]
        x2all[...] = x + x
        sq_c[...] = jnp.sum(x * x, axis=1, keepdims=True)
        v = lab_ref[...]
        m = jnp.max(v, axis=1, keepdims=True)
        iota = jax.lax.broadcasted_iota(jnp.int32, v.shape, 1)
        lab_c[0:B, :] = jnp.min(jnp.where(v == m, iota, C), axis=1,
                                keepdims=True)
        lab_c[B:N, :] = pk_ref[...]
        lab_c[N:NP_, :] = -(jax.lax.broadcasted_iota(jnp.int32, (NPAD, 1), 0)
                            + 1)
        for k in range(NT):
            sq_r[k:k + 1, :] = jnp.transpose(
                sq_c[k * TILE:(k + 1) * TILE, :])
            lab_r[k:k + 1, :] = jnp.transpose(
                lab_c[k * TILE:(k + 1) * TILE, :])

    bi = tiles_ref[0, t]
    bj = tiles_ref[1, t]
    ri = pl.ds(bi * TILE, TILE)
    rj = pl.ds(bj * TILE, TILE)

    xi = xall[ri, :]
    xj2 = x2all[rj, :]
    dot2 = jax.lax.dot_general(xi, xj2, (((1,), (1,)), ((), ())),
                               preferred_element_type=jnp.float32)
    sqi = sq_c[ri, :]
    sqj = sq_r[pl.ds(bj, 1), :]
    raw = (sqi + sqj) - dot2
    d2 = jnp.maximum(raw, 0.0)
    q = jnp.clip(raw, 1e-12, MARGIN * MARGIN)
    r = MARGIN - q * jax.lax.rsqrt(q)
    same = lab_c[ri, :] == lab_r[pl.ds(bj, 1), :]
    base = jnp.where(same, d2, r * r)
    s = jnp.sum(base)

    scale = jnp.where(bi == bj, 0.5, 1.0)
    out_ref[0, 0] += s * scale

    @pl.when(t == NUM_TILES - 1)
    def _finish():
        out_ref[0, 0] = out_ref[0, 0] * (1.0 / N_PAIRS)


def kernel(embeddings, labels, prototypes, proto_keys):
    tiles = jnp.asarray(_PAIRS)
    pk2d = proto_keys.astype(jnp.int32)[:, None]

    out = pl.pallas_call(
        _body,
        grid_spec=pltpu.PrefetchScalarGridSpec(
            num_scalar_prefetch=1,
            grid=(NUM_TILES,),
            in_specs=[
                pl.BlockSpec((B, D), lambda t, tiles: (0, 0)),
                pl.BlockSpec((B, C), lambda t, tiles: (0, 0)),
                pl.BlockSpec((P, D), lambda t, tiles: (0, 0)),
                pl.BlockSpec((P, 1), lambda t, tiles: (0, 0)),
            ],
            out_specs=pl.BlockSpec(memory_space=pltpu.SMEM),
            scratch_shapes=[
                pltpu.VMEM((NP_, D), jnp.float32),
                pltpu.VMEM((NP_, D), jnp.float32),
                pltpu.VMEM((NP_, 1), jnp.float32),
                pltpu.VMEM((NT, TILE), jnp.float32),
                pltpu.VMEM((NP_, 1), jnp.int32),
                pltpu.VMEM((NT, TILE), jnp.int32),
            ],
        ),
        out_shape=jax.ShapeDtypeStruct((1, 1), jnp.float32),
    )(tiles, embeddings, labels, prototypes, pk2d)
    return out[0, 0]


# speedup vs baseline: 1.9615x; 1.0187x over previous
import jax
import jax.numpy as jnp
import numpy as np
from jax.experimental import pallas as pl
from jax.experimental.pallas import tpu as pltpu

B, D, C, P = 2048, 128, 200, 200
N = B + P
TILE = 768
NP_ = 2304
NT = NP_ // TILE
NPAD = NP_ - N
MARGIN = 1.0
N_PAIRS = float(N * (N - 1) // 2)

_PAIRS = np.array([(i, j) for i in range(NT) for j in range(i, NT)],
                  dtype=np.int32).T
NUM_TILES = _PAIRS.shape[1]


def _body(tiles_ref, emb_ref, lab_ref, proto_ref, pk_ref,
          out_ref, xall, x2all, sq_c, sq_r, lab_c, lab_r):
    t = pl.program_id(0)

    @pl.when(t == 0)
    def _prep():
        out_ref[0, 0] = 0.0
        xall[0:B, :] = emb_ref[...]
        xall[B:N, :] = proto_ref[...]
        padv = 2.0 * (jax.lax.broadcasted_iota(jnp.int32, (NPAD, D), 0)
                      .astype(jnp.float32) + 1.0)
        xall[N:NP_, :] = padv
        x = xall[...]
        x2all[...] = x + x
        sq_c[...] = jnp.sum(x * x, axis=1, keepdims=True)
        v = lab_ref[...]
        m = jnp.max(v, axis=1, keepdims=True)
        iota = jax.lax.broadcasted_iota(jnp.int32, v.shape, 1)
        lab_c[0:B, :] = jnp.min(jnp.where(v == m, iota, C), axis=1,
                                keepdims=True)
        lab_c[B:N, :] = pk_ref[...]
        lab_c[N:NP_, :] = -(jax.lax.broadcasted_iota(jnp.int32, (NPAD, 1), 0)
                            + 1)
        for k in range(NT):
            sq_r[k:k + 1, :] = jnp.transpose(
                sq_c[k * TILE:(k + 1) * TILE, :])
            lab_r[k:k + 1, :] = jnp.transpose(
                lab_c[k * TILE:(k + 1) * TILE, :])

    bi = tiles_ref[0, t]
    bj = tiles_ref[1, t]
    ri = pl.ds(bi * TILE, TILE)
    rj = pl.ds(bj * TILE, TILE)

    xi = xall[ri, :]
    xj2 = x2all[rj, :]
    dot2 = jax.lax.dot_general(xi, xj2, (((1,), (1,)), ((), ())),
                               preferred_element_type=jnp.float32)
    sqi = sq_c[ri, :]
    sqj = sq_r[pl.ds(bj, 1), :]
    li = lab_c[ri, :]
    lj = lab_r[pl.ds(bj, 1), :]

    CH = 16
    eps = jnp.float32(1e-12)
    one = jnp.float32(MARGIN * MARGIN)
    acc = jnp.zeros((CH, TILE), jnp.float32)
    for k in range(TILE // CH):
        sl = slice(k * CH, (k + 1) * CH)
        raw = (sqi[sl, :] + sqj) - dot2[sl, :]
        q = jax.lax.clamp(eps, raw, one)
        r = MARGIN - q * jax.lax.rsqrt(q)
        same = li[sl, :] == lj
        acc = acc + jnp.where(same, raw, r * r)
    s = jnp.sum(acc)

    scale = jnp.where(bi == bj, 0.5, 1.0)
    out_ref[0, 0] += s * scale

    @pl.when(t == NUM_TILES - 1)
    def _finish():
        out_ref[0, 0] = out_ref[0, 0] * (1.0 / N_PAIRS)


def kernel(embeddings, labels, prototypes, proto_keys):
    tiles = jnp.asarray(_PAIRS)
    pk2d = proto_keys.astype(jnp.int32)[:, None]

    out = pl.pallas_call(
        _body,
        grid_spec=pltpu.PrefetchScalarGridSpec(
            num_scalar_prefetch=1,
            grid=(NUM_TILES,),
            in_specs=[
                pl.BlockSpec((B, D), lambda t, tiles: (0, 0)),
                pl.BlockSpec((B, C), lambda t, tiles: (0, 0)),
                pl.BlockSpec((P, D), lambda t, tiles: (0, 0)),
                pl.BlockSpec((P, 1), lambda t, tiles: (0, 0)),
            ],
            out_specs=pl.BlockSpec(memory_space=pltpu.SMEM),
            scratch_shapes=[
                pltpu.VMEM((NP_, D), jnp.float32),
                pltpu.VMEM((NP_, D), jnp.float32),
                pltpu.VMEM((NP_, 1), jnp.float32),
                pltpu.VMEM((NT, TILE), jnp.float32),
                pltpu.VMEM((NP_, 1), jnp.int32),
                pltpu.VMEM((NT, TILE), jnp.int32),
            ],
        ),
        out_shape=jax.ShapeDtypeStruct((1, 1), jnp.float32),
    )(tiles, embeddings, labels, prototypes, pk2d)
    return out[0, 0]
